# baseline (speedup 1.0000x reference)
import dataclasses
import functools

import jax
import jax.numpy as jnp
from jax import lax
from jax.experimental import pallas as pl
from jax.experimental.pallas import tpu as pltpu
from jax.experimental.pallas import tpu_sc as plsc

BATCH = 16384
DIM = 64
NENT = 1000000
NC = 2
NS = 16
NW = NC * NS
BPW = BATCH // NW
CHUNK = 128
NCHUNK = BPW // CHUNK
GPC = CHUNK // 16

LBLK = 512
NBLK = (NENT + LBLK - 1) // LBLK


def _tpose_body(x_ref, y_ref):
    z = x_ref[...].T
    y_ref[...] = jnp.concatenate([z[: LBLK // 2, :], z[LBLK // 2 :, :]], axis=1)


def _pair_view(table_t):
    return pl.pallas_call(
        _tpose_body,
        grid=(NBLK,),
        in_specs=[pl.BlockSpec((DIM, LBLK), lambda i: (0, i))],
        out_specs=pl.BlockSpec((LBLK // 2, 128), lambda i: (i, 0)),
        out_shape=jax.ShapeDtypeStruct((NBLK * (LBLK // 2), 2 * DIM), jnp.float32),
    )(table_t)


def _sc_body(hi_hbm, ri_hbm, ti_hbm, ent_hbm, rel_hbm, out_hbm,
             hi_v, ri_v, ti_v, pidx_h, pidx_r, pidx_t,
             hdat, rdat, tdat, out_v, sem):
    wid = lax.axis_index("s") * NC + lax.axis_index("c")
    base = wid * BPW

    pltpu.sync_copy(hi_hbm.at[pl.ds(base, BPW)], hi_v)
    pltpu.sync_copy(ri_hbm.at[pl.ds(base, BPW)], ri_v)
    pltpu.sync_copy(ti_hbm.at[pl.ds(base, BPW)], ti_v)

    lanes = lax.iota(jnp.int32, 16)

    @pl.loop(0, NCHUNK)
    def _(c):
        @pl.loop(0, GPC)
        def _(g):
            sl = pl.ds(c * CHUNK + g * 16, 16)
            dsl = pl.ds(g * 16, 16)
            pidx_h[dsl] = ((hi_v[sl] >> 9) << 8) + (hi_v[sl] & 255)
            pidx_r[dsl] = ((ri_v[sl] >> 9) << 8) + (ri_v[sl] & 255)
            pidx_t[dsl] = ((ti_v[sl] >> 9) << 8) + (ti_v[sl] & 255)

        ch = pltpu.async_copy(ent_hbm.at[pidx_h], hdat, sem)
        cr = pltpu.async_copy(rel_hbm.at[pidx_r], rdat, sem)
        ct = pltpu.async_copy(ent_hbm.at[pidx_t], tdat, sem)
        ch.wait()
        cr.wait()
        ct.wait()

        @pl.loop(0, GPC)
        def _(g):
            sl = pl.ds(c * CHUNK + g * 16, 16)
            oh = ((hi_v[sl] >> 8) & 1) << 6
            orr = ((ri_v[sl] >> 8) & 1) << 6
            ot = ((ti_v[sl] >> 8) & 1) << 6
            j16 = g * 16 + lanes
            acc = jnp.zeros((16,), jnp.float32)
            for d in range(DIM):
                h = plsc.load_gather(hdat, [j16, oh + d])
                r = plsc.load_gather(rdat, [j16, orr + d])
                t = plsc.load_gather(tdat, [j16, ot + d])
                acc = acc + h * r * t
            out_v[pl.ds(c * CHUNK + g * 16, 16)] = acc

    pltpu.sync_copy(out_v, out_hbm.at[pl.ds(base, BPW)])


@jax.jit
def kernel(sample, entity_embedding, relation_embedding):
    hi = sample[:, 0].astype(jnp.int32)
    ri = sample[:, 1].astype(jnp.int32)
    ti = sample[:, 2].astype(jnp.int32)
    ent2 = _pair_view(entity_embedding.T)
    rel2 = _pair_view(relation_embedding.T)

    mesh = plsc.VectorSubcoreMesh(core_axis_name="c", subcore_axis_name="s")
    cp = pltpu.CompilerParams(use_tc_tiling_on_sc=True)
    if "needs_layout_passes" in pltpu.CompilerParams.__dataclass_fields__:
        cp = dataclasses.replace(cp, needs_layout_passes=False)
    run = pl.kernel(
        _sc_body,
        out_type=jax.ShapeDtypeStruct((BATCH,), jnp.float32),
        mesh=mesh,
        scratch_types=[
            pltpu.VMEM((BPW,), jnp.int32),
            pltpu.VMEM((BPW,), jnp.int32),
            pltpu.VMEM((BPW,), jnp.int32),
            pltpu.VMEM((CHUNK,), jnp.int32),
            pltpu.VMEM((CHUNK,), jnp.int32),
            pltpu.VMEM((CHUNK,), jnp.int32),
            pltpu.VMEM((CHUNK, 2 * DIM), jnp.float32),
            pltpu.VMEM((CHUNK, 2 * DIM), jnp.float32),
            pltpu.VMEM((CHUNK, 2 * DIM), jnp.float32),
            pltpu.VMEM((BPW,), jnp.float32),
            pltpu.SemaphoreType.DMA,
        ],
        compiler_params=cp,
    )
    score = run(hi, ri, ti, ent2, rel2)
    return score.reshape(BATCH, 1)


# speedup vs baseline: 2.5909x; 2.5909x over previous
import dataclasses
import functools

import jax
import jax.numpy as jnp
from jax import lax
from jax.experimental import pallas as pl
from jax.experimental.pallas import tpu as pltpu
from jax.experimental.pallas import tpu_sc as plsc

BATCH = 16384
DIM = 64
NENT = 1000000
NC = 2
NS = 16
NW = NC * NS
BPW = BATCH // NW
CHUNK = 128
NCHUNK = BPW // CHUNK
GPC = CHUNK // 16

LBLK = 2048
NBLK = (NENT + LBLK - 1) // LBLK


def _tpose_body(x_ref, y_ref):
    y_ref[:, :DIM] = x_ref[:, : LBLK // 2].T
    y_ref[:, DIM:] = x_ref[:, LBLK // 2 :].T


def _pair_view(table_t):
    return pl.pallas_call(
        _tpose_body,
        grid=(NBLK,),
        in_specs=[pl.BlockSpec((DIM, LBLK), lambda i: (0, i))],
        out_specs=pl.BlockSpec((LBLK // 2, 128), lambda i: (i, 0)),
        out_shape=jax.ShapeDtypeStruct((NBLK * (LBLK // 2), 2 * DIM), jnp.float32),
    )(table_t)


def _sc_body(hi_hbm, ri_hbm, ti_hbm, ent_hbm, rel_hbm, out_hbm,
             hi_v, ri_v, ti_v, pidx_h, pidx_r, pidx_t,
             hdat, rdat, tdat, out_v, sem):
    wid = lax.axis_index("s") * NC + lax.axis_index("c")
    base = wid * BPW

    pltpu.sync_copy(hi_hbm.at[pl.ds(base, BPW)], hi_v)
    pltpu.sync_copy(ri_hbm.at[pl.ds(base, BPW)], ri_v)
    pltpu.sync_copy(ti_hbm.at[pl.ds(base, BPW)], ti_v)

    lanes = lax.iota(jnp.int32, 16)

    @pl.loop(0, NCHUNK)
    def _(c):
        @pl.loop(0, GPC)
        def _(g):
            sl = pl.ds(c * CHUNK + g * 16, 16)
            dsl = pl.ds(g * 16, 16)
            pidx_h[dsl] = ((hi_v[sl] >> 11) << 10) + (hi_v[sl] & 1023)
            pidx_r[dsl] = ((ri_v[sl] >> 11) << 10) + (ri_v[sl] & 1023)
            pidx_t[dsl] = ((ti_v[sl] >> 11) << 10) + (ti_v[sl] & 1023)

        ch = pltpu.async_copy(ent_hbm.at[pidx_h], hdat, sem)
        cr = pltpu.async_copy(rel_hbm.at[pidx_r], rdat, sem)
        ct = pltpu.async_copy(ent_hbm.at[pidx_t], tdat, sem)
        ch.wait()
        cr.wait()
        ct.wait()

        @pl.loop(0, GPC)
        def _(g):
            sl = pl.ds(c * CHUNK + g * 16, 16)
            oh = ((hi_v[sl] >> 10) & 1) << 6
            orr = ((ri_v[sl] >> 10) & 1) << 6
            ot = ((ti_v[sl] >> 10) & 1) << 6
            j16 = g * 16 + lanes
            acc = jnp.zeros((16,), jnp.float32)
            for d in range(DIM):
                h = plsc.load_gather(hdat, [j16, oh + d])
                r = plsc.load_gather(rdat, [j16, orr + d])
                t = plsc.load_gather(tdat, [j16, ot + d])
                acc = acc + h * r * t
            out_v[pl.ds(c * CHUNK + g * 16, 16)] = acc

    pltpu.sync_copy(out_v, out_hbm.at[pl.ds(base, BPW)])


@jax.jit
def kernel(sample, entity_embedding, relation_embedding):
    hi = sample[:, 0].astype(jnp.int32)
    ri = sample[:, 1].astype(jnp.int32)
    ti = sample[:, 2].astype(jnp.int32)
    ent2 = _pair_view(entity_embedding.T)
    rel2 = _pair_view(relation_embedding.T)

    mesh = plsc.VectorSubcoreMesh(core_axis_name="c", subcore_axis_name="s")
    cp = pltpu.CompilerParams(use_tc_tiling_on_sc=True)
    if "needs_layout_passes" in pltpu.CompilerParams.__dataclass_fields__:
        cp = dataclasses.replace(cp, needs_layout_passes=False)
    run = pl.kernel(
        _sc_body,
        out_type=jax.ShapeDtypeStruct((BATCH,), jnp.float32),
        mesh=mesh,
        scratch_types=[
            pltpu.VMEM((BPW,), jnp.int32),
            pltpu.VMEM((BPW,), jnp.int32),
            pltpu.VMEM((BPW,), jnp.int32),
            pltpu.VMEM((CHUNK,), jnp.int32),
            pltpu.VMEM((CHUNK,), jnp.int32),
            pltpu.VMEM((CHUNK,), jnp.int32),
            pltpu.VMEM((CHUNK, 2 * DIM), jnp.float32),
            pltpu.VMEM((CHUNK, 2 * DIM), jnp.float32),
            pltpu.VMEM((CHUNK, 2 * DIM), jnp.float32),
            pltpu.VMEM((BPW,), jnp.float32),
            pltpu.SemaphoreType.DMA,
        ],
        compiler_params=cp,
    )
    score = run(hi, ri, ti, ent2, rel2)
    return score.reshape(BATCH, 1)


# speedup vs baseline: 4.2400x; 1.6365x over previous
import dataclasses
import functools

import jax
import jax.numpy as jnp
from jax import lax
from jax.experimental import pallas as pl
from jax.experimental.pallas import tpu as pltpu
from jax.experimental.pallas import tpu_sc as plsc

BATCH = 16384
DIM = 64
NENT = 1000000
NC = 2
NS = 16
NW = NC * NS
BPW = BATCH // NW
CHUNK = 128
NCHUNK = BPW // CHUNK
GPC = CHUNK // 16

LBLK = 8192
NBLK = (NENT + LBLK - 1) // LBLK


def _tpose_body(x_ref, y_ref):
    y_ref[:, :DIM] = x_ref[:, : LBLK // 2].T
    y_ref[:, DIM:] = x_ref[:, LBLK // 2 :].T


def _pair_view(table_t):
    return pl.pallas_call(
        _tpose_body,
        grid=(NBLK,),
        in_specs=[pl.BlockSpec((DIM, LBLK), lambda i: (0, i))],
        out_specs=pl.BlockSpec((LBLK // 2, 128), lambda i: (i, 0)),
        out_shape=jax.ShapeDtypeStruct((NBLK * (LBLK // 2), 2 * DIM), jnp.float32),
    )(table_t)


def _sc_body(hi_hbm, ri_hbm, ti_hbm, ent_hbm, rel_hbm, out_hbm,
             hi_v, ri_v, ti_v, pidx_h, pidx_r, pidx_t,
             hdat, rdat, tdat, out_v, sem):
    wid = lax.axis_index("s") * NC + lax.axis_index("c")
    base = wid * BPW

    pltpu.sync_copy(hi_hbm.at[pl.ds(base, BPW)], hi_v)
    pltpu.sync_copy(ri_hbm.at[pl.ds(base, BPW)], ri_v)
    pltpu.sync_copy(ti_hbm.at[pl.ds(base, BPW)], ti_v)

    lanes = lax.iota(jnp.int32, 16)

    @pl.loop(0, NCHUNK)
    def _(c):
        @pl.loop(0, GPC)
        def _(g):
            sl = pl.ds(c * CHUNK + g * 16, 16)
            dsl = pl.ds(g * 16, 16)
            pidx_h[dsl] = ((hi_v[sl] >> 13) << 12) + (hi_v[sl] & 4095)
            pidx_r[dsl] = ((ri_v[sl] >> 13) << 12) + (ri_v[sl] & 4095)
            pidx_t[dsl] = ((ti_v[sl] >> 13) << 12) + (ti_v[sl] & 4095)

        ch = pltpu.async_copy(ent_hbm.at[pidx_h], hdat, sem)
        cr = pltpu.async_copy(rel_hbm.at[pidx_r], rdat, sem)
        ct = pltpu.async_copy(ent_hbm.at[pidx_t], tdat, sem)
        ch.wait()
        cr.wait()
        ct.wait()

        @pl.loop(0, GPC)
        def _(g):
            sl = pl.ds(c * CHUNK + g * 16, 16)
            oh = ((hi_v[sl] >> 12) & 1) << 6
            orr = ((ri_v[sl] >> 12) & 1) << 6
            ot = ((ti_v[sl] >> 12) & 1) << 6
            j16 = g * 16 + lanes
            acc = jnp.zeros((16,), jnp.float32)
            for d in range(DIM):
                h = plsc.load_gather(hdat, [j16, oh + d])
                r = plsc.load_gather(rdat, [j16, orr + d])
                t = plsc.load_gather(tdat, [j16, ot + d])
                acc = acc + h * r * t
            out_v[pl.ds(c * CHUNK + g * 16, 16)] = acc

    pltpu.sync_copy(out_v, out_hbm.at[pl.ds(base, BPW)])


@jax.jit
def kernel(sample, entity_embedding, relation_embedding):
    hi = sample[:, 0].astype(jnp.int32)
    ri = sample[:, 1].astype(jnp.int32)
    ti = sample[:, 2].astype(jnp.int32)
    ent2 = _pair_view(entity_embedding.T)
    rel2 = _pair_view(relation_embedding.T)

    mesh = plsc.VectorSubcoreMesh(core_axis_name="c", subcore_axis_name="s")
    cp = pltpu.CompilerParams(use_tc_tiling_on_sc=True)
    if "needs_layout_passes" in pltpu.CompilerParams.__dataclass_fields__:
        cp = dataclasses.replace(cp, needs_layout_passes=False)
    run = pl.kernel(
        _sc_body,
        out_type=jax.ShapeDtypeStruct((BATCH,), jnp.float32),
        mesh=mesh,
        scratch_types=[
            pltpu.VMEM((BPW,), jnp.int32),
            pltpu.VMEM((BPW,), jnp.int32),
            pltpu.VMEM((BPW,), jnp.int32),
            pltpu.VMEM((CHUNK,), jnp.int32),
            pltpu.VMEM((CHUNK,), jnp.int32),
            pltpu.VMEM((CHUNK,), jnp.int32),
            pltpu.VMEM((CHUNK, 2 * DIM), jnp.float32),
            pltpu.VMEM((CHUNK, 2 * DIM), jnp.float32),
            pltpu.VMEM((CHUNK, 2 * DIM), jnp.float32),
            pltpu.VMEM((BPW,), jnp.float32),
            pltpu.SemaphoreType.DMA,
        ],
        compiler_params=cp,
    )
    score = run(hi, ri, ti, ent2, rel2)
    return score.reshape(BATCH, 1)


# speedup vs baseline: 4.7718x; 1.1254x over previous
import dataclasses
import functools

import jax
import jax.numpy as jnp
from jax import lax
from jax.experimental import pallas as pl
from jax.experimental.pallas import tpu as pltpu
from jax.experimental.pallas import tpu_sc as plsc

BATCH = 16384
DIM = 64
NENT = 1000000
NC = 2
NS = 16
NW = NC * NS
BPW = BATCH // NW
CHUNK = 128
NCHUNK = BPW // CHUNK
GPC = CHUNK // 16

LBLK = 16384
NBLK = (NENT + LBLK - 1) // LBLK


def _tpose_body(x_ref, y_ref):
    y_ref[:, :DIM] = x_ref[:, : LBLK // 2].T
    y_ref[:, DIM:] = x_ref[:, LBLK // 2 :].T


def _pair_view(table_t):
    return pl.pallas_call(
        _tpose_body,
        grid=(NBLK,),
        in_specs=[pl.BlockSpec((DIM, LBLK), lambda i: (0, i))],
        out_specs=pl.BlockSpec((LBLK // 2, 128), lambda i: (i, 0)),
        out_shape=jax.ShapeDtypeStruct((NBLK * (LBLK // 2), 2 * DIM), jnp.float32),
    )(table_t)


def _sc_body(hi_hbm, ri_hbm, ti_hbm, ent_hbm, rel_hbm, out_hbm,
             hi_v, ri_v, ti_v, pidx_h, pidx_r, pidx_t,
             hdat, rdat, tdat, out_v, sem):
    wid = lax.axis_index("s") * NC + lax.axis_index("c")
    base = wid * BPW

    pltpu.sync_copy(hi_hbm.at[pl.ds(base, BPW)], hi_v)
    pltpu.sync_copy(ri_hbm.at[pl.ds(base, BPW)], ri_v)
    pltpu.sync_copy(ti_hbm.at[pl.ds(base, BPW)], ti_v)

    lanes = lax.iota(jnp.int32, 16)

    @pl.loop(0, NCHUNK)
    def _(c):
        @pl.loop(0, GPC)
        def _(g):
            sl = pl.ds(c * CHUNK + g * 16, 16)
            dsl = pl.ds(g * 16, 16)
            pidx_h[dsl] = ((hi_v[sl] >> 14) << 13) + (hi_v[sl] & 8191)
            pidx_r[dsl] = ((ri_v[sl] >> 14) << 13) + (ri_v[sl] & 8191)
            pidx_t[dsl] = ((ti_v[sl] >> 14) << 13) + (ti_v[sl] & 8191)

        ch = pltpu.async_copy(ent_hbm.at[pidx_h], hdat, sem)
        cr = pltpu.async_copy(rel_hbm.at[pidx_r], rdat, sem)
        ct = pltpu.async_copy(ent_hbm.at[pidx_t], tdat, sem)
        ch.wait()
        cr.wait()
        ct.wait()

        @pl.loop(0, GPC)
        def _(g):
            sl = pl.ds(c * CHUNK + g * 16, 16)
            oh = ((hi_v[sl] >> 13) & 1) << 6
            orr = ((ri_v[sl] >> 13) & 1) << 6
            ot = ((ti_v[sl] >> 13) & 1) << 6
            j16 = g * 16 + lanes
            acc = jnp.zeros((16,), jnp.float32)
            for d in range(DIM):
                h = plsc.load_gather(hdat, [j16, oh + d])
                r = plsc.load_gather(rdat, [j16, orr + d])
                t = plsc.load_gather(tdat, [j16, ot + d])
                acc = acc + h * r * t
            out_v[pl.ds(c * CHUNK + g * 16, 16)] = acc

    pltpu.sync_copy(out_v, out_hbm.at[pl.ds(base, BPW)])


@jax.jit
def kernel(sample, entity_embedding, relation_embedding):
    hi = sample[:, 0].astype(jnp.int32)
    ri = sample[:, 1].astype(jnp.int32)
    ti = sample[:, 2].astype(jnp.int32)
    ent2 = _pair_view(entity_embedding.T)
    rel2 = _pair_view(relation_embedding.T)

    mesh = plsc.VectorSubcoreMesh(core_axis_name="c", subcore_axis_name="s")
    cp = pltpu.CompilerParams(use_tc_tiling_on_sc=True)
    if "needs_layout_passes" in pltpu.CompilerParams.__dataclass_fields__:
        cp = dataclasses.replace(cp, needs_layout_passes=False)
    run = pl.kernel(
        _sc_body,
        out_type=jax.ShapeDtypeStruct((BATCH,), jnp.float32),
        mesh=mesh,
        scratch_types=[
            pltpu.VMEM((BPW,), jnp.int32),
            pltpu.VMEM((BPW,), jnp.int32),
            pltpu.VMEM((BPW,), jnp.int32),
            pltpu.VMEM((CHUNK,), jnp.int32),
            pltpu.VMEM((CHUNK,), jnp.int32),
            pltpu.VMEM((CHUNK,), jnp.int32),
            pltpu.VMEM((CHUNK, 2 * DIM), jnp.float32),
            pltpu.VMEM((CHUNK, 2 * DIM), jnp.float32),
            pltpu.VMEM((CHUNK, 2 * DIM), jnp.float32),
            pltpu.VMEM((BPW,), jnp.float32),
            pltpu.SemaphoreType.DMA,
        ],
        compiler_params=cp,
    )
    score = run(hi, ri, ti, ent2, rel2)
    return score.reshape(BATCH, 1)


# speedup vs baseline: 5.0418x; 1.0566x over previous
import dataclasses
import functools

import jax
import jax.numpy as jnp
from jax import lax
from jax.experimental import pallas as pl
from jax.experimental.pallas import tpu as pltpu
from jax.experimental.pallas import tpu_sc as plsc

BATCH = 16384
DIM = 64
NENT = 1000000
NC = 2
NS = 16
NW = NC * NS
BPW = BATCH // NW
CHUNK = 128
NCHUNK = BPW // CHUNK
GPC = CHUNK // 16

LBLK = 32768
NBLK = (NENT + LBLK - 1) // LBLK


def _tpose_body(x_ref, y_ref):
    y_ref[:, :DIM] = x_ref[:, : LBLK // 2].T
    y_ref[:, DIM:] = x_ref[:, LBLK // 2 :].T


def _pair_view(table_t):
    return pl.pallas_call(
        _tpose_body,
        grid=(NBLK,),
        in_specs=[pl.BlockSpec((DIM, LBLK), lambda i: (0, i))],
        out_specs=pl.BlockSpec((LBLK // 2, 128), lambda i: (i, 0)),
        out_shape=jax.ShapeDtypeStruct((NBLK * (LBLK // 2), 2 * DIM), jnp.float32),
    )(table_t)


def _sc_body(hi_hbm, ri_hbm, ti_hbm, ent_hbm, rel_hbm, out_hbm,
             hi_v, ri_v, ti_v, pidx_h, pidx_r, pidx_t,
             hdat, rdat, tdat, out_v, sem):
    wid = lax.axis_index("s") * NC + lax.axis_index("c")
    base = wid * BPW

    pltpu.sync_copy(hi_hbm.at[pl.ds(base, BPW)], hi_v)
    pltpu.sync_copy(ri_hbm.at[pl.ds(base, BPW)], ri_v)
    pltpu.sync_copy(ti_hbm.at[pl.ds(base, BPW)], ti_v)

    lanes = lax.iota(jnp.int32, 16)

    @pl.loop(0, NCHUNK)
    def _(c):
        @pl.loop(0, GPC)
        def _(g):
            sl = pl.ds(c * CHUNK + g * 16, 16)
            dsl = pl.ds(g * 16, 16)
            pidx_h[dsl] = ((hi_v[sl] >> 15) << 14) + (hi_v[sl] & 16383)
            pidx_r[dsl] = ((ri_v[sl] >> 15) << 14) + (ri_v[sl] & 16383)
            pidx_t[dsl] = ((ti_v[sl] >> 15) << 14) + (ti_v[sl] & 16383)

        ch = pltpu.async_copy(ent_hbm.at[pidx_h], hdat, sem)
        cr = pltpu.async_copy(rel_hbm.at[pidx_r], rdat, sem)
        ct = pltpu.async_copy(ent_hbm.at[pidx_t], tdat, sem)
        ch.wait()
        cr.wait()
        ct.wait()

        @pl.loop(0, GPC)
        def _(g):
            sl = pl.ds(c * CHUNK + g * 16, 16)
            oh = ((hi_v[sl] >> 14) & 1) << 6
            orr = ((ri_v[sl] >> 14) & 1) << 6
            ot = ((ti_v[sl] >> 14) & 1) << 6
            j16 = g * 16 + lanes
            acc = jnp.zeros((16,), jnp.float32)
            for d in range(DIM):
                h = plsc.load_gather(hdat, [j16, oh + d])
                r = plsc.load_gather(rdat, [j16, orr + d])
                t = plsc.load_gather(tdat, [j16, ot + d])
                acc = acc + h * r * t
            out_v[pl.ds(c * CHUNK + g * 16, 16)] = acc

    pltpu.sync_copy(out_v, out_hbm.at[pl.ds(base, BPW)])


@jax.jit
def kernel(sample, entity_embedding, relation_embedding):
    hi = sample[:, 0].astype(jnp.int32)
    ri = sample[:, 1].astype(jnp.int32)
    ti = sample[:, 2].astype(jnp.int32)
    ent2 = _pair_view(entity_embedding.T)
    rel2 = _pair_view(relation_embedding.T)

    mesh = plsc.VectorSubcoreMesh(core_axis_name="c", subcore_axis_name="s")
    cp = pltpu.CompilerParams(use_tc_tiling_on_sc=True)
    if "needs_layout_passes" in pltpu.CompilerParams.__dataclass_fields__:
        cp = dataclasses.replace(cp, needs_layout_passes=False)
    run = pl.kernel(
        _sc_body,
        out_type=jax.ShapeDtypeStruct((BATCH,), jnp.float32),
        mesh=mesh,
        scratch_types=[
            pltpu.VMEM((BPW,), jnp.int32),
            pltpu.VMEM((BPW,), jnp.int32),
            pltpu.VMEM((BPW,), jnp.int32),
            pltpu.VMEM((CHUNK,), jnp.int32),
            pltpu.VMEM((CHUNK,), jnp.int32),
            pltpu.VMEM((CHUNK,), jnp.int32),
            pltpu.VMEM((CHUNK, 2 * DIM), jnp.float32),
            pltpu.VMEM((CHUNK, 2 * DIM), jnp.float32),
            pltpu.VMEM((CHUNK, 2 * DIM), jnp.float32),
            pltpu.VMEM((BPW,), jnp.float32),
            pltpu.SemaphoreType.DMA,
        ],
        compiler_params=cp,
    )
    score = run(hi, ri, ti, ent2, rel2)
    return score.reshape(BATCH, 1)


# speedup vs baseline: 5.1061x; 1.0127x over previous
import dataclasses
import functools

import jax
import jax.numpy as jnp
from jax import lax
from jax.experimental import pallas as pl
from jax.experimental.pallas import tpu as pltpu
from jax.experimental.pallas import tpu_sc as plsc

BATCH = 16384
DIM = 64
NENT = 1000000
NC = 2
NS = 16
NW = NC * NS
BPW = BATCH // NW
CHUNK = 128
NCHUNK = BPW // CHUNK
GPC = CHUNK // 16

LBLK = 32768
NBLK = (NENT + LBLK - 1) // LBLK


def _tpose_body(x_ref, y_ref):
    y_ref[:, :DIM] = x_ref[:, : LBLK // 2].T
    y_ref[:, DIM:] = x_ref[:, LBLK // 2 :].T


def _pair_view(table_t):
    return pl.pallas_call(
        _tpose_body,
        grid=(NBLK,),
        in_specs=[pl.BlockSpec((DIM, LBLK), lambda i: (0, i))],
        out_specs=pl.BlockSpec((LBLK // 2, 128), lambda i: (i, 0)),
        out_shape=jax.ShapeDtypeStruct((NBLK * (LBLK // 2), 2 * DIM), jnp.float32),
    )(table_t)


def _sc_body(hi_hbm, ri_hbm, ti_hbm, ent_hbm, rel_hbm, out_hbm,
             hi_v, ri_v, ti_v,
             ph0, pr0, pt0, ph1, pr1, pt1,
             hd0, rd0, td0, hd1, rd1, td1,
             out_v, sem0, sem1):
    wid = lax.axis_index("s") * NC + lax.axis_index("c")
    base = wid * BPW

    pltpu.sync_copy(hi_hbm.at[pl.ds(base, BPW)], hi_v)
    pltpu.sync_copy(ri_hbm.at[pl.ds(base, BPW)], ri_v)
    pltpu.sync_copy(ti_hbm.at[pl.ds(base, BPW)], ti_v)

    pidx = ((ph0, pr0, pt0), (ph1, pr1, pt1))
    dat = ((hd0, rd0, td0), (hd1, rd1, td1))
    sems = (sem0, sem1)
    lanes = lax.iota(jnp.int32, 16)

    def build_and_fire(c, slot):
        ph, pr, pt = pidx[slot]

        @pl.loop(0, GPC)
        def _(g):
            sl = pl.ds(c * CHUNK + g * 16, 16)
            dsl = pl.ds(g * 16, 16)
            ph[dsl] = ((hi_v[sl] >> 15) << 14) + (hi_v[sl] & 16383)
            pr[dsl] = ((ri_v[sl] >> 15) << 14) + (ri_v[sl] & 16383)
            pt[dsl] = ((ti_v[sl] >> 15) << 14) + (ti_v[sl] & 16383)

        hd, rd, td = dat[slot]
        pltpu.async_copy(ent_hbm.at[ph], hd, sems[slot])
        pltpu.async_copy(rel_hbm.at[pr], rd, sems[slot])
        pltpu.async_copy(ent_hbm.at[pt], td, sems[slot])

    def drain(slot):
        ph, pr, pt = pidx[slot]
        hd, rd, td = dat[slot]
        pltpu.make_async_copy(ent_hbm.at[ph], hd, sems[slot]).wait()
        pltpu.make_async_copy(rel_hbm.at[pr], rd, sems[slot]).wait()
        pltpu.make_async_copy(ent_hbm.at[pt], td, sems[slot]).wait()

    def compute(c, slot):
        hd, rd, td = dat[slot]

        @pl.loop(0, GPC)
        def _(g):
            sl = pl.ds(c * CHUNK + g * 16, 16)
            oh = ((hi_v[sl] >> 14) & 1) << 6
            orr = ((ri_v[sl] >> 14) & 1) << 6
            ot = ((ti_v[sl] >> 14) & 1) << 6
            j16 = g * 16 + lanes
            acc = jnp.zeros((16,), jnp.float32)
            for d in range(DIM):
                h = plsc.load_gather(hd, [j16, oh + d])
                r = plsc.load_gather(rd, [j16, orr + d])
                t = plsc.load_gather(td, [j16, ot + d])
                acc = acc + h * r * t
            out_v[pl.ds(c * CHUNK + g * 16, 16)] = acc

    build_and_fire(0, 0)
    for c in range(NCHUNK):
        slot = c & 1
        if c + 1 < NCHUNK:
            build_and_fire(c + 1, 1 - slot)
        drain(slot)
        compute(c, slot)

    pltpu.sync_copy(out_v, out_hbm.at[pl.ds(base, BPW)])


@jax.jit
def kernel(sample, entity_embedding, relation_embedding):
    hi = sample[:, 0].astype(jnp.int32)
    ri = sample[:, 1].astype(jnp.int32)
    ti = sample[:, 2].astype(jnp.int32)
    ent2 = _pair_view(entity_embedding.T)
    rel2 = _pair_view(relation_embedding.T)

    mesh = plsc.VectorSubcoreMesh(core_axis_name="c", subcore_axis_name="s")
    cp = pltpu.CompilerParams(use_tc_tiling_on_sc=True)
    if "needs_layout_passes" in pltpu.CompilerParams.__dataclass_fields__:
        cp = dataclasses.replace(cp, needs_layout_passes=False)
    run = pl.kernel(
        _sc_body,
        out_type=jax.ShapeDtypeStruct((BATCH,), jnp.float32),
        mesh=mesh,
        scratch_types=[
            pltpu.VMEM((BPW,), jnp.int32),
            pltpu.VMEM((BPW,), jnp.int32),
            pltpu.VMEM((BPW,), jnp.int32),
            pltpu.VMEM((CHUNK,), jnp.int32),
            pltpu.VMEM((CHUNK,), jnp.int32),
            pltpu.VMEM((CHUNK,), jnp.int32),
            pltpu.VMEM((CHUNK,), jnp.int32),
            pltpu.VMEM((CHUNK,), jnp.int32),
            pltpu.VMEM((CHUNK,), jnp.int32),
            pltpu.VMEM((CHUNK, 2 * DIM), jnp.float32),
            pltpu.VMEM((CHUNK, 2 * DIM), jnp.float32),
            pltpu.VMEM((CHUNK, 2 * DIM), jnp.float32),
            pltpu.VMEM((CHUNK, 2 * DIM), jnp.float32),
            pltpu.VMEM((CHUNK, 2 * DIM), jnp.float32),
            pltpu.VMEM((CHUNK, 2 * DIM), jnp.float32),
            pltpu.VMEM((BPW,), jnp.float32),
            pltpu.SemaphoreType.DMA,
            pltpu.SemaphoreType.DMA,
        ],
        compiler_params=cp,
    )
    score = run(hi, ri, ti, ent2, rel2)
    return score.reshape(BATCH, 1)


# speedup vs baseline: 5.2138x; 1.0211x over previous
import dataclasses
import functools

import jax
import jax.numpy as jnp
from jax import lax
from jax.experimental import pallas as pl
from jax.experimental.pallas import tpu as pltpu
from jax.experimental.pallas import tpu_sc as plsc

BATCH = 16384
DIM = 64
NENT = 1000000
NC = 2
NS = 16
NW = NC * NS
BPW = BATCH // NW
CHUNK = 128
NCHUNK = BPW // CHUNK
GPC = CHUNK // 16

LBLK = 32768
NBLK = (NENT + LBLK - 1) // LBLK
HALF = LBLK // 2


def _tpose_body(x_ref, y_ref):
    y_ref[:, :DIM] = x_ref[:, :HALF].T
    y_ref[:, DIM:] = x_ref[:, HALF:].T


def _pair_view(table_t):
    return pl.pallas_call(
        _tpose_body,
        grid=(NBLK,),
        in_specs=[pl.BlockSpec((DIM, LBLK), lambda i: (0, i))],
        out_specs=pl.BlockSpec((HALF, 128), lambda i: (i, 0)),
        out_shape=jax.ShapeDtypeStruct((NBLK * HALF, 2 * DIM), jnp.float32),
    )(table_t)


def _pair_idx(r):
    return ((r >> 15) << 14) + (r & 16383)


def _pair_off(r):
    return ((r >> 14) & 1) << 6


def _rel_body(ri_hbm, rel_hbm, out_hbm, ri_v, p0, p1, d0, d1, rbuf, sem0, sem1):
    wid = lax.axis_index("s") * NC + lax.axis_index("c")
    base = wid * BPW

    pltpu.sync_copy(ri_hbm.at[pl.ds(base, BPW)], ri_v)

    pidx = (p0, p1)
    dat = (d0, d1)
    sems = (sem0, sem1)
    lanes = lax.iota(jnp.int32, 16)

    def fire(c, slot):
        p = pidx[slot]

        @pl.loop(0, GPC)
        def _(g):
            p[pl.ds(g * 16, 16)] = _pair_idx(ri_v[pl.ds(c * CHUNK + g * 16, 16)])

        pltpu.async_copy(rel_hbm.at[p], dat[slot], sems[slot])

    def drain(slot):
        pltpu.make_async_copy(rel_hbm.at[pidx[slot]], dat[slot], sems[slot]).wait()

    fire(0, 0)
    for c in range(NCHUNK):
        slot = c & 1
        if c + 1 < NCHUNK:
            fire(c + 1, 1 - slot)
        drain(slot)
        rd = dat[slot]

        @pl.loop(0, GPC)
        def _(g):
            sl = pl.ds(c * CHUNK + g * 16, 16)
            orr = _pair_off(ri_v[sl])
            j16 = g * 16 + lanes
            for d in range(DIM):
                rbuf[d, sl] = plsc.load_gather(rd, [j16, orr + d])

    pltpu.sync_copy(rbuf, out_hbm.at[:, pl.ds(base, BPW)])


def _score_body(hi_hbm, ti_hbm, ent_hbm, rrow_hbm, out_hbm,
                hi_v, ti_v, ph0, pt0, ph1, pt1,
                hd0, td0, hd1, td1, rv, out_v, sem0, sem1):
    wid = lax.axis_index("s") * NC + lax.axis_index("c")
    base = wid * BPW

    pltpu.sync_copy(hi_hbm.at[pl.ds(base, BPW)], hi_v)
    pltpu.sync_copy(ti_hbm.at[pl.ds(base, BPW)], ti_v)
    pltpu.sync_copy(rrow_hbm.at[:, pl.ds(base, BPW)], rv)

    pidx = ((ph0, pt0), (ph1, pt1))
    dat = ((hd0, td0), (hd1, td1))
    sems = (sem0, sem1)
    lanes = lax.iota(jnp.int32, 16)

    def fire(c, slot):
        ph, pt = pidx[slot]

        @pl.loop(0, GPC)
        def _(g):
            sl = pl.ds(c * CHUNK + g * 16, 16)
            dsl = pl.ds(g * 16, 16)
            ph[dsl] = _pair_idx(hi_v[sl])
            pt[dsl] = _pair_idx(ti_v[sl])

        hd, td = dat[slot]
        pltpu.async_copy(ent_hbm.at[ph], hd, sems[slot])
        pltpu.async_copy(ent_hbm.at[pt], td, sems[slot])

    def drain(slot):
        ph, pt = pidx[slot]
        hd, td = dat[slot]
        pltpu.make_async_copy(ent_hbm.at[ph], hd, sems[slot]).wait()
        pltpu.make_async_copy(ent_hbm.at[pt], td, sems[slot]).wait()

    fire(0, 0)
    for c in range(NCHUNK):
        slot = c & 1
        if c + 1 < NCHUNK:
            fire(c + 1, 1 - slot)
        drain(slot)
        hd, td = dat[slot]

        @pl.loop(0, GPC)
        def _(g):
            sl = pl.ds(c * CHUNK + g * 16, 16)
            oh = _pair_off(hi_v[sl])
            ot = _pair_off(ti_v[sl])
            j16 = g * 16 + lanes
            acc = jnp.zeros((16,), jnp.float32)
            for d in range(DIM):
                h = plsc.load_gather(hd, [j16, oh + d])
                t = plsc.load_gather(td, [j16, ot + d])
                acc = acc + h * t * rv[d, sl]
            out_v[pl.ds(c * CHUNK + g * 16, 16)] = acc

    pltpu.sync_copy(out_v, out_hbm.at[pl.ds(base, BPW)])


@jax.jit
def kernel(sample, entity_embedding, relation_embedding):
    hi = sample[:, 0].astype(jnp.int32)
    ri = sample[:, 1].astype(jnp.int32)
    ti = sample[:, 2].astype(jnp.int32)

    mesh = plsc.VectorSubcoreMesh(core_axis_name="c", subcore_axis_name="s")
    cp = pltpu.CompilerParams(use_tc_tiling_on_sc=True)
    if "needs_layout_passes" in pltpu.CompilerParams.__dataclass_fields__:
        cp = dataclasses.replace(cp, needs_layout_passes=False)

    rel2 = _pair_view(relation_embedding.T)
    k_rel = pl.kernel(
        _rel_body,
        out_type=jax.ShapeDtypeStruct((DIM, BATCH), jnp.float32),
        mesh=mesh,
        scratch_types=[
            pltpu.VMEM((BPW,), jnp.int32),
            pltpu.VMEM((CHUNK,), jnp.int32),
            pltpu.VMEM((CHUNK,), jnp.int32),
            pltpu.VMEM((CHUNK, 2 * DIM), jnp.float32),
            pltpu.VMEM((CHUNK, 2 * DIM), jnp.float32),
            pltpu.VMEM((DIM, BPW), jnp.float32),
            pltpu.SemaphoreType.DMA,
            pltpu.SemaphoreType.DMA,
        ],
        compiler_params=cp,
    )
    rrow = k_rel(ri, rel2)

    ent2 = _pair_view(entity_embedding.T)
    k_score = pl.kernel(
        _score_body,
        out_type=jax.ShapeDtypeStruct((BATCH,), jnp.float32),
        mesh=mesh,
        scratch_types=[
            pltpu.VMEM((BPW,), jnp.int32),
            pltpu.VMEM((BPW,), jnp.int32),
            pltpu.VMEM((CHUNK,), jnp.int32),
            pltpu.VMEM((CHUNK,), jnp.int32),
            pltpu.VMEM((CHUNK,), jnp.int32),
            pltpu.VMEM((CHUNK,), jnp.int32),
            pltpu.VMEM((CHUNK, 2 * DIM), jnp.float32),
            pltpu.VMEM((CHUNK, 2 * DIM), jnp.float32),
            pltpu.VMEM((CHUNK, 2 * DIM), jnp.float32),
            pltpu.VMEM((CHUNK, 2 * DIM), jnp.float32),
            pltpu.VMEM((DIM, BPW), jnp.float32),
            pltpu.VMEM((BPW,), jnp.float32),
            pltpu.SemaphoreType.DMA,
            pltpu.SemaphoreType.DMA,
        ],
        compiler_params=cp,
    )
    score = k_score(hi, ti, ent2, rrow)
    return score.reshape(BATCH, 1)


# speedup vs baseline: 5.2359x; 1.0042x over previous
import dataclasses

import jax
import jax.numpy as jnp
from jax import lax
from jax.experimental import pallas as pl
from jax.experimental.pallas import tpu as pltpu
from jax.experimental.pallas import tpu_sc as plsc

BATCH = 16384
DIM = 64
NENT = 1000000
NC = 2
NS = 16
NW = NC * NS
BPW = BATCH // NW
CHUNK = 128
NCHUNK = BPW // CHUNK
GPC = CHUNK // 16

LBLK = 32768
NBLK = (NENT + LBLK - 1) // LBLK
HALF = LBLK // 2


def _tpose_body(x_ref, y_ref):
    y_ref[:, :DIM] = x_ref[:, :HALF].T
    y_ref[:, DIM:] = x_ref[:, HALF:].T


def _pair_view(table_t):
    return pl.pallas_call(
        _tpose_body,
        grid=(NBLK,),
        in_specs=[pl.BlockSpec((DIM, LBLK), lambda i: (0, i))],
        out_specs=pl.BlockSpec((HALF, 128), lambda i: (i, 0)),
        out_shape=jax.ShapeDtypeStruct((NBLK * HALF, 2 * DIM), jnp.float32),
    )(table_t)


def _pair_idx(r):
    return ((r >> 15) << 14) + (r & 16383)


def _pair_off(r):
    return ((r >> 14) & 1) << 6


def _rel_body(ri_hbm, rel_hbm, out_hbm, ri_v, p0, p1, d0, d1, rbuf, sem0, sem1):
    wid = lax.axis_index("s") * NC + lax.axis_index("c")
    base = wid * BPW

    pltpu.sync_copy(ri_hbm.at[pl.ds(base, BPW)], ri_v)

    pidx = (p0, p1)
    dat = (d0, d1)
    sems = (sem0, sem1)
    lanes = lax.iota(jnp.int32, 16)

    def fire(c, slot):
        p = pidx[slot]

        @pl.loop(0, GPC)
        def _(g):
            p[pl.ds(g * 16, 16)] = _pair_idx(ri_v[pl.ds(c * CHUNK + g * 16, 16)])

        pltpu.async_copy(rel_hbm.at[p], dat[slot], sems[slot])

    def drain(slot):
        pltpu.make_async_copy(rel_hbm.at[pidx[slot]], dat[slot], sems[slot]).wait()

    fire(0, 0)
    for c in range(NCHUNK):
        slot = c & 1
        if c + 1 < NCHUNK:
            fire(c + 1, 1 - slot)
        drain(slot)
        rd = dat[slot]

        @pl.loop(0, GPC)
        def _(g):
            sl = pl.ds(c * CHUNK + g * 16, 16)
            orr = _pair_off(ri_v[sl])
            j16 = g * 16 + lanes
            for d in range(DIM):
                rbuf[d, sl] = plsc.load_gather(rd, [j16, orr + d])

    pltpu.sync_copy(rbuf, out_hbm.at[:, pl.ds(base, BPW)])


def _score_body(hi_hbm, ti_hbm, ent_hbm, rrow_hbm, out_hbm,
                hi_v, ti_v, ph0, pt0, ph1, pt1,
                hd0, td0, hd1, td1, rv, out_v, sem0, sem1):
    wid = lax.axis_index("s") * NC + lax.axis_index("c")
    base = wid * BPW

    pltpu.sync_copy(hi_hbm.at[pl.ds(base, BPW)], hi_v)
    pltpu.sync_copy(ti_hbm.at[pl.ds(base, BPW)], ti_v)

    pidx = ((ph0, pt0), (ph1, pt1))
    dat = ((hd0, td0), (hd1, td1))
    sems = (sem0, sem1)
    lanes = lax.iota(jnp.int32, 16)

    def fire(c, slot):
        ph, pt = pidx[slot]

        @pl.loop(0, GPC)
        def _(g):
            sl = pl.ds(c * CHUNK + g * 16, 16)
            dsl = pl.ds(g * 16, 16)
            ph[dsl] = _pair_idx(hi_v[sl])
            pt[dsl] = _pair_idx(ti_v[sl])

        hd, td = dat[slot]
        pltpu.async_copy(ent_hbm.at[ph], hd, sems[slot])
        pltpu.async_copy(ent_hbm.at[pt], td, sems[slot])

    def drain(slot):
        ph, pt = pidx[slot]
        hd, td = dat[slot]
        pltpu.make_async_copy(ent_hbm.at[ph], hd, sems[slot]).wait()
        pltpu.make_async_copy(ent_hbm.at[pt], td, sems[slot]).wait()

    fire(0, 0)
    pltpu.sync_copy(rrow_hbm.at[:, pl.ds(base, BPW)], rv)
    for c in range(NCHUNK):
        slot = c & 1
        if c + 1 < NCHUNK:
            fire(c + 1, 1 - slot)
        drain(slot)
        hd, td = dat[slot]

        @pl.loop(0, GPC)
        def _(g):
            sl = pl.ds(c * CHUNK + g * 16, 16)
            oh = _pair_off(hi_v[sl])
            ot = _pair_off(ti_v[sl])
            j16 = g * 16 + lanes
            acc = jnp.zeros((16,), jnp.float32)
            for d in range(DIM):
                h = plsc.load_gather(hd, [j16, oh + d])
                t = plsc.load_gather(td, [j16, ot + d])
                acc = acc + h * t * rv[d, sl]
            out_v[pl.ds(c * CHUNK + g * 16, 16)] = acc

    pltpu.sync_copy(out_v, out_hbm.at[pl.ds(base, BPW)])


@jax.jit
def kernel(sample, entity_embedding, relation_embedding):
    hi = sample[:, 0].astype(jnp.int32)
    ri = sample[:, 1].astype(jnp.int32)
    ti = sample[:, 2].astype(jnp.int32)

    mesh = plsc.VectorSubcoreMesh(core_axis_name="c", subcore_axis_name="s")
    cp = pltpu.CompilerParams(use_tc_tiling_on_sc=True)
    if "needs_layout_passes" in pltpu.CompilerParams.__dataclass_fields__:
        cp = dataclasses.replace(cp, needs_layout_passes=False)

    rel2 = _pair_view(relation_embedding.T)
    k_rel = pl.kernel(
        _rel_body,
        out_type=jax.ShapeDtypeStruct((DIM, BATCH), jnp.float32),
        mesh=mesh,
        scratch_types=[
            pltpu.VMEM((BPW,), jnp.int32),
            pltpu.VMEM((CHUNK,), jnp.int32),
            pltpu.VMEM((CHUNK,), jnp.int32),
            pltpu.VMEM((CHUNK, 2 * DIM), jnp.float32),
            pltpu.VMEM((CHUNK, 2 * DIM), jnp.float32),
            pltpu.VMEM((DIM, BPW), jnp.float32),
            pltpu.SemaphoreType.DMA,
            pltpu.SemaphoreType.DMA,
        ],
        compiler_params=cp,
    )
    rrow = k_rel(ri, rel2)

    ent2 = _pair_view(entity_embedding.T)
    k_score = pl.kernel(
        _score_body,
        out_type=jax.ShapeDtypeStruct((BATCH,), jnp.float32),
        mesh=mesh,
        scratch_types=[
            pltpu.VMEM((BPW,), jnp.int32),
            pltpu.VMEM((BPW,), jnp.int32),
            pltpu.VMEM((CHUNK,), jnp.int32),
            pltpu.VMEM((CHUNK,), jnp.int32),
            pltpu.VMEM((CHUNK,), jnp.int32),
            pltpu.VMEM((CHUNK,), jnp.int32),
            pltpu.VMEM((CHUNK, 2 * DIM), jnp.float32),
            pltpu.VMEM((CHUNK, 2 * DIM), jnp.float32),
            pltpu.VMEM((CHUNK, 2 * DIM), jnp.float32),
            pltpu.VMEM((CHUNK, 2 * DIM), jnp.float32),
            pltpu.VMEM((DIM, BPW), jnp.float32),
            pltpu.VMEM((BPW,), jnp.float32),
            pltpu.SemaphoreType.DMA,
            pltpu.SemaphoreType.DMA,
        ],
        compiler_params=cp,
    )
    score = k_score(hi, ti, ent2, rrow)
    return score.reshape(BATCH, 1)
